# Initial kernel scaffold; baseline (speedup 1.0000x reference)
import jax
import jax.numpy as jnp
from jax.experimental import pallas as pl


def kernel(latent, params):
    raise NotImplementedError("write your pallas kernel here")


# speedup vs baseline: 2.4119x; 2.4119x over previous
import functools

import jax
import jax.numpy as jnp
from jax import lax
from jax.experimental import pallas as pl
from jax.experimental.pallas import tpu as pltpu
from jax.experimental.pallas import tpu_sc as plsc

B = 64
LATENT_DIM = 1024
HIDDEN_DIM = 1024
SET_CH = 128
COS_CH = 64
POINT_NUM = 8192
MAX_N = 256

L = 16
NC, NS = 2, 16
NW = NC * NS
ROWS_PER_W = B // NW
PV = POINT_NUM // L
HB = 256
HBV = HB // L
K = MAX_N

_HIMASK = [0, -16777216, -65536, -256]
_SHIFT = [24, 16, 8, 0]


def _head_body(lat_ref, w1_ref, b1_ref, w2_ref, b2_ref, w3_ref, b3_ref,
               w4_ref, b4_ref, ang_ref, w1b_ref, ob1_ref,
               probs_ref, latpart_ref):
    lat = lat_ref[...]
    h = jnp.maximum(jnp.dot(lat, w1_ref[...],
                            preferred_element_type=jnp.float32) + b1_ref[...], 0.0)
    h = jnp.maximum(jnp.dot(h, w2_ref[...],
                            preferred_element_type=jnp.float32) + b2_ref[...], 0.0)
    h = jnp.maximum(jnp.dot(h, w3_ref[...],
                            preferred_element_type=jnp.float32) + b3_ref[...], 0.0)
    a = jnp.dot(h, w4_ref[...], preferred_element_type=jnp.float32) + b4_ref[...]
    nrm = jnp.sqrt(jnp.sum(a * a, axis=1, keepdims=True))
    a = a / (nrm + 1e-5)
    cos = lax.dot_general(a, ang_ref[...], (((1,), (1,)), ((), ())),
                          preferred_element_type=jnp.float32)
    m = jnp.max(cos, axis=1, keepdims=True)
    e = jnp.exp(cos - m)
    probs_ref[...] = e / jnp.sum(e, axis=1, keepdims=True)
    latpart_ref[...] = jnp.dot(lat, w1b_ref[...],
                               preferred_element_type=jnp.float32) + ob1_ref[...]


def _head_call(latent, p, w1b):
    return pl.pallas_call(
        _head_body,
        out_shape=[
            jax.ShapeDtypeStruct((B, POINT_NUM), jnp.float32),
            jax.ShapeDtypeStruct((B, HIDDEN_DIM + SET_CH), jnp.float32),
        ],
    )(latent,
      p["mlp_w1"], p["mlp_b1"].reshape(1, -1),
      p["mlp_w2"], p["mlp_b2"].reshape(1, -1),
      p["mlp_w3"], p["mlp_b3"].reshape(1, -1),
      p["mlp_w4"], p["mlp_b4"].reshape(1, -1),
      p["angles_params"], w1b, p["out_b1"].reshape(1, -1))


def _sc_row(r, probs_hbm, points_hbm, svals_hbm, sel_hbm,
            row_ref, hist_ref, total_ref, cumt_ref, cval_ref, cidx_ref,
            cidx2_ref, rows_ref, sem):
    lanes = lax.iota(jnp.int32, L)
    ones = jnp.ones((L,), jnp.int32)
    lane_base = lanes * HB

    pltpu.sync_copy(probs_hbm.at[r], row_ref)

    thr = jnp.int32(0)
    above = jnp.int32(0)
    for p_i in range(4):
        shift = _SHIFT[p_i]
        himask = jnp.int32(_HIMASK[p_i])

        def zero_body(i, _):
            hist_ref[pl.ds(i * L, L)] = jnp.zeros((L,), jnp.int32)
            return 0
        lax.fori_loop(0, HB, zero_body, 0)

        tmask = thr & himask

        def hist_body(j, _, shift=shift, himask=himask, tmask=tmask):
            v = row_ref[pl.ds(j * L, L)]
            u = plsc.bitcast(v, jnp.int32)
            cand = (u & himask) == tmask
            bucket = lax.shift_right_logical(u, shift) & 0xFF
            plsc.addupdate_scatter(hist_ref, [lane_base + bucket], ones,
                                   mask=cand)
            return 0
        lax.fori_loop(0, PV, hist_body, 0)

        def red_body(c, _):
            acc = jnp.zeros((L,), jnp.int32)
            for l in range(L):
                acc = acc + hist_ref[pl.ds(l * HB + c * L, L)]
            total_ref[pl.ds(c * L, L)] = acc
            return 0
        lax.fori_loop(0, HBV, red_body, 0)

        def scan_body(i, carry, above=above):
            cumc, tc = carry
            c = HBV - 1 - i
            t = total_ref[pl.ds(c * L, L)]
            trev = lax.rev(t, (0,))
            cs = plsc.cumsum(trev) + cumc
            cumt_ref[pl.ds(c * L, L)] = lax.rev(cs, (0,))
            cond = (above + cs) >= K
            npos = jnp.sum(cond.astype(jnp.int32))
            return (jnp.max(cs), tc + npos)
        _, tc = lax.fori_loop(0, HBV, scan_body,
                              (jnp.int32(0), jnp.int32(0)))
        bstar = tc - 1
        splat = jnp.zeros((L,), jnp.int32) + bstar
        mcnt = jnp.max(plsc.load_gather(total_ref, [splat]))
        cge = jnp.max(plsc.load_gather(cumt_ref, [splat]))
        thr = thr | lax.shift_left(bstar, shift)
        above = above + cge - mcnt

    def comp_body(j, carry):
        off, eq_left = carry
        v = row_ref[pl.ds(j * L, L)]
        u = plsc.bitcast(v, jnp.int32)
        gt = u > thr
        eq = u == thr
        eqc = plsc.cumsum(eq.astype(jnp.int32))
        take_eq = eq & (eqc <= eq_left)
        keep = gt | take_eq
        kc = plsc.cumsum(keep.astype(jnp.int32))
        addr = off + kc - 1
        plsc.store_scatter(cval_ref, [addr], v, mask=keep)
        plsc.store_scatter(cidx_ref, [addr], j * L + lanes, mask=keep)
        return (off + jnp.max(kc), eq_left - jnp.sum(take_eq.astype(jnp.int32)))
    lax.fori_loop(0, PV, comp_body, (jnp.int32(0), K - above))

    for i in range(K // L):
        sk, sv = plsc.sort_key_val(cval_ref[pl.ds(i * L, L)],
                                   cidx_ref[pl.ds(i * L, L)], descending=True)
        cval_ref[pl.ds(i * L, L)] = sk
        cidx_ref[pl.ds(i * L, L)] = sv
    for m in (1, 2, 4, 8):
        for s0 in range(0, K // L, 2 * m):
            ks = [cval_ref[pl.ds((s0 + i) * L, L)] for i in range(2 * m)]
            vs = [cidx_ref[pl.ds((s0 + i) * L, L)] for i in range(2 * m)]
            ks[m:] = [lax.rev(x, (0,)) for x in reversed(ks[m:])]
            vs[m:] = [lax.rev(x, (0,)) for x in reversed(vs[m:])]
            step = m
            while step >= 1:
                for base in range(0, 2 * m, 2 * step):
                    for i in range(base, base + step):
                        ak, bk = ks[i], ks[i + step]
                        av, bv = vs[i], vs[i + step]
                        c = ak >= bk
                        ks[i] = jnp.where(c, ak, bk)
                        ks[i + step] = jnp.where(c, bk, ak)
                        vs[i] = jnp.where(c, av, bv)
                        vs[i + step] = jnp.where(c, bv, av)
                step //= 2
            for i in range(2 * m):
                sk, sv = plsc.sort_key_val(ks[i], vs[i], descending=True)
                cval_ref[pl.ds((s0 + i) * L, L)] = sk
                cidx_ref[pl.ds((s0 + i) * L, L)] = sv

    for c in range(2):
        for i in range(128 // L):
            cidx2_ref[c, pl.ds(i * L, L)] = cidx_ref[pl.ds(c * 128 + i * L, L)]

    d0 = pltpu.async_copy(points_hbm.at[cidx2_ref.at[0]],
                          rows_ref.at[pl.ds(0, 128)], sem)
    d1 = pltpu.async_copy(points_hbm.at[cidx2_ref.at[1]],
                          rows_ref.at[pl.ds(128, 128)], sem)
    d0.wait()
    d1.wait()

    pltpu.sync_copy(cval_ref.at[pl.ds(0, K)], svals_hbm.at[r])
    pltpu.sync_copy(rows_ref, sel_hbm.at[r])


def _topk_call(probs, points):
    mesh = plsc.VectorSubcoreMesh(core_axis_name="c", subcore_axis_name="s",
                                  num_cores=NC, num_subcores=NS)

    @functools.partial(
        pl.kernel,
        out_type=[
            jax.ShapeDtypeStruct((B, K), jnp.float32),
            jax.ShapeDtypeStruct((B, K, SET_CH), jnp.float32),
        ],
        mesh=mesh,
        compiler_params=pltpu.CompilerParams(needs_layout_passes=False),
        scratch_types=[
            pltpu.VMEM((POINT_NUM,), jnp.float32),
            pltpu.VMEM((HB * L,), jnp.int32),
            pltpu.VMEM((HB,), jnp.int32),
            pltpu.VMEM((HB,), jnp.int32),
            pltpu.VMEM((K + L,), jnp.float32),
            pltpu.VMEM((K + L,), jnp.int32),
            pltpu.VMEM((2, 128), jnp.int32),
            pltpu.VMEM((K, SET_CH), jnp.float32),
            pltpu.SemaphoreType.DMA,
        ],
    )
    def _k(probs_hbm, points_hbm, svals_hbm, sel_hbm,
           row_ref, hist_ref, total_ref, cumt_ref, cval_ref, cidx_ref,
           cidx2_ref, rows_ref, sem):
        wid = lax.axis_index("s") * NC + lax.axis_index("c")

        def row_body(rr, _):
            _sc_row(wid * ROWS_PER_W + rr, probs_hbm, points_hbm,
                    svals_hbm, sel_hbm, row_ref, hist_ref, total_ref,
                    cumt_ref, cval_ref, cidx_ref, cidx2_ref, rows_ref, sem)
            return 0
        lax.fori_loop(0, ROWS_PER_W, row_body, 0)

    return _k(probs, points)


_BB = 8


def _tail_body(sel_ref, sv_ref, lp_ref, l1w_ref, l1b_ref, l2w_ref, l2b_ref,
               w1a_ref, w2_ref, b2_ref, w3_ref, b3_ref, out_ref):
    w1a = w1a_ref[...]
    w2 = w2_ref[...]
    b2 = b2_ref[...]
    w3 = w3_ref[...]
    b3 = b3_ref[...]
    l1w = l1w_ref[...]
    l1b = l1b_ref[...]
    l2w = l2w_ref[...]
    l2b = l2b_ref[...]
    for i in range(_BB):
        scaled = MAX_N * sv_ref[i]
        alpha = scaled * l1w + l1b
        beta = scaled * l2w + l2b
        mod = alpha * sel_ref[i] + beta
        t = jnp.dot(mod, w1a, preferred_element_type=jnp.float32)
        h = jnp.maximum(t + lp_ref[i], 0.0)
        h2 = jnp.maximum(jnp.dot(h, w2, preferred_element_type=jnp.float32)
                         + b2, 0.0)
        out_ref[i] = jnp.dot(h2, w3, preferred_element_type=jnp.float32) + b3


def _tail_call(sel, svals, latpart, p, w1a):
    d_cat = SET_CH + LATENT_DIM
    return pl.pallas_call(
        _tail_body,
        grid=(B // _BB,),
        in_specs=[
            pl.BlockSpec((_BB, K, SET_CH), lambda i: (i, 0, 0)),
            pl.BlockSpec((_BB, K, 1), lambda i: (i, 0, 0)),
            pl.BlockSpec((_BB, 1, d_cat), lambda i: (i, 0, 0)),
            pl.BlockSpec((1, SET_CH), lambda i: (0, 0)),
            pl.BlockSpec((1, SET_CH), lambda i: (0, 0)),
            pl.BlockSpec((1, SET_CH), lambda i: (0, 0)),
            pl.BlockSpec((1, SET_CH), lambda i: (0, 0)),
            pl.BlockSpec((SET_CH, d_cat), lambda i: (0, 0)),
            pl.BlockSpec((d_cat, SET_CH), lambda i: (0, 0)),
            pl.BlockSpec((1, SET_CH), lambda i: (0, 0)),
            pl.BlockSpec((SET_CH, SET_CH), lambda i: (0, 0)),
            pl.BlockSpec((1, SET_CH), lambda i: (0, 0)),
        ],
        out_specs=pl.BlockSpec((_BB, K, SET_CH), lambda i: (i, 0, 0)),
        out_shape=jax.ShapeDtypeStruct((B, K, SET_CH), jnp.float32),
    )(sel, svals.reshape(B, K, 1), latpart.reshape(B, 1, d_cat),
      p["lin1_w"].reshape(1, -1), p["lin1_b"].reshape(1, -1),
      p["lin2_w"].reshape(1, -1), p["lin2_b"].reshape(1, -1),
      w1a, p["out_w2"], p["out_b2"].reshape(1, -1),
      p["out_w3"], p["out_b3"].reshape(1, -1))


def kernel(latent, params):
    p = params
    w1a = p["out_w1"][:SET_CH]
    w1b = p["out_w1"][SET_CH:]
    probs, latpart = _head_call(latent, p, w1b)
    svals, sel = _topk_call(probs, p["points"])
    out = _tail_call(sel, svals, latpart, p, w1a)
    mask = jnp.ones((B, MAX_N, 1), dtype=jnp.float32)
    n_arr = jnp.full((B,), MAX_N, dtype=jnp.int32)
    return (out, mask, n_arr)
